# baseline (speedup 1.0000x reference)
import functools

import jax
import jax.numpy as jnp
from jax.experimental import pallas as pl

N_SHELVES = 50
E = 32
NSTEP = 4


def _leaky(x):
    return jnp.maximum(x, 0.01 * x)


def _fused_body(s_ref, x_ref, w1_ref, w2_ref, b1_ref, b2_ref, out_ref):
    w1 = w1_ref[...]
    w2 = w2_ref[...]
    b1 = b1_ref[...]
    b2 = b2_ref[...]
    for k in range(NSTEP):
        m = s_ref[k] == N_SHELVES
        xs = x_ref[k]
        pre = jax.lax.dot(w1, xs,
                          preferred_element_type=jnp.float32) + b1
        h = _leaky(jnp.where(m, pre[:E], pre[E:]))
        o = jax.lax.dot(w2, h,
                        preferred_element_type=jnp.float32) + b2
        out_ref[k] = _leaky(jnp.where(m, o[:E], o[E:]))


@functools.partial(jax.jit, static_argnames=())
def kernel(state, x, vw1, vb1, vw2, vb2, sw1, sb1, sw2, sb2):
    B, NV, FEAT = x.shape

    xt = jnp.transpose(x, (1, 2, 0))
    st = jnp.transpose(state, (1, 2, 0))

    w1v = jnp.zeros((E, FEAT), jnp.float32).at[:, 0].set(vw1[0])
    w1 = jnp.concatenate([w1v, sw1.T], axis=0)
    w2 = jnp.concatenate([vw2.T, sw2.T], axis=0)
    b1 = jnp.concatenate([vb1, sb1]).reshape(2 * E, 1)
    b2 = jnp.concatenate([vb2, sb2]).reshape(2 * E, 1)

    grid = (NV // NSTEP,)

    outt = pl.pallas_call(
        _fused_body,
        grid=grid,
        in_specs=[
            pl.BlockSpec((NSTEP, 1, B), lambda i: (i, 0, 0)),
            pl.BlockSpec((NSTEP, FEAT, B), lambda i: (i, 0, 0)),
            pl.BlockSpec((2 * E, FEAT), lambda i: (0, 0)),
            pl.BlockSpec((2 * E, FEAT), lambda i: (0, 0)),
            pl.BlockSpec((2 * E, 1), lambda i: (0, 0)),
            pl.BlockSpec((2 * E, 1), lambda i: (0, 0)),
        ],
        out_specs=pl.BlockSpec((NSTEP, E, B), lambda i: (i, 0, 0)),
        out_shape=jax.ShapeDtypeStruct((NV, E, B), jnp.float32),
    )(st, xt, w1, w2, b1, b2)

    return jnp.transpose(outt, (2, 0, 1))


# speedup vs baseline: 1.1025x; 1.1025x over previous
import functools

import jax
import jax.numpy as jnp
from jax.experimental import pallas as pl

N_SHELVES = 50
E = 32
NSTEP = 10


def _leaky(x):
    return jnp.maximum(x, 0.01 * x)


def _fused_body(s_ref, x_ref, w1_ref, w2_ref, b1_ref, b2_ref, out_ref):
    w1 = w1_ref[...]
    w2 = w2_ref[...]
    b1 = b1_ref[...]
    b2 = b2_ref[...]
    for k in range(NSTEP):
        m = s_ref[k] == N_SHELVES
        xs = x_ref[k]
        pre = jax.lax.dot(w1, xs,
                          preferred_element_type=jnp.float32) + b1
        h = _leaky(jnp.where(m, pre[:E], pre[E:]))
        o = jax.lax.dot(w2, h,
                        preferred_element_type=jnp.float32) + b2
        out_ref[k] = _leaky(jnp.where(m, o[:E], o[E:]))


@functools.partial(jax.jit, static_argnames=())
def kernel(state, x, vw1, vb1, vw2, vb2, sw1, sb1, sw2, sb2):
    B, NV, FEAT = x.shape

    xt = jnp.transpose(x, (1, 2, 0))
    st = jnp.transpose(state, (1, 2, 0))

    w1v = jnp.zeros((E, FEAT), jnp.float32).at[:, 0].set(vw1[0])
    w1 = jnp.concatenate([w1v, sw1.T], axis=0)
    w2 = jnp.concatenate([vw2.T, sw2.T], axis=0)
    b1 = jnp.concatenate([vb1, sb1]).reshape(2 * E, 1)
    b2 = jnp.concatenate([vb2, sb2]).reshape(2 * E, 1)

    grid = (NV // NSTEP,)

    outt = pl.pallas_call(
        _fused_body,
        grid=grid,
        in_specs=[
            pl.BlockSpec((NSTEP, 1, B), lambda i: (i, 0, 0)),
            pl.BlockSpec((NSTEP, FEAT, B), lambda i: (i, 0, 0)),
            pl.BlockSpec((2 * E, FEAT), lambda i: (0, 0)),
            pl.BlockSpec((2 * E, FEAT), lambda i: (0, 0)),
            pl.BlockSpec((2 * E, 1), lambda i: (0, 0)),
            pl.BlockSpec((2 * E, 1), lambda i: (0, 0)),
        ],
        out_specs=pl.BlockSpec((NSTEP, E, B), lambda i: (i, 0, 0)),
        out_shape=jax.ShapeDtypeStruct((NV, E, B), jnp.float32),
    )(st, xt, w1, w2, b1, b2)

    return jnp.transpose(outt, (2, 0, 1))
